# Initial kernel scaffold; baseline (speedup 1.0000x reference)
import jax
import jax.numpy as jnp
from jax.experimental import pallas as pl


def kernel(inputs):
    raise NotImplementedError("write your pallas kernel here")


# speedup vs baseline: 137.3948x; 137.3948x over previous
import jax
import jax.numpy as jnp
from jax.experimental import pallas as pl
from jax.experimental.pallas import tpu as pltpu

_CHUNK = 512

_SORT8_PAIRS = [(0, 1), (2, 3), (4, 5), (6, 7),
                (0, 2), (1, 3), (4, 6), (5, 7),
                (1, 2), (5, 6), (0, 4), (3, 7),
                (1, 5), (2, 6),
                (1, 4), (3, 6),
                (2, 4), (3, 5),
                (3, 4)]

_BITONIC_STAGES = [[(0, 4), (1, 5), (2, 6), (3, 7)],
                   [(0, 2), (1, 3), (4, 6), (5, 7)],
                   [(0, 1), (2, 3), (4, 5), (6, 7)]]


def _cex(v, i, j):
    hi = jnp.maximum(v[i], v[j])
    lo = jnp.minimum(v[i], v[j])
    v[i] = hi
    v[j] = lo


def _sort8(vs):
    vs = list(vs)
    for i, j in _SORT8_PAIRS:
        _cex(vs, i, j)
    return vs


def _merge8(a, b):
    v = [jnp.maximum(a[i], b[7 - i]) for i in range(8)]
    for stage in _BITONIC_STAGES:
        for i, j in stage:
            _cex(v, i, j)
    return v


def _tree_topk(vs):
    vs = _sort8(vs)
    r = vs[0].shape[0]
    while r > 1:
        h = r // 2
        vs = _merge8([v[:h] for v in vs], [v[h:] for v in vs])
        r = h
    return vs


def _leaf_kernel(x_ref, o_ref):
    x = x_ref[0]
    g = _CHUNK // 8
    vs = _tree_topk([x[g * j:g * (j + 1)] for j in range(8)])
    o_ref[0, 0] = jnp.concatenate(vs, axis=0)


def _combine_kernel(x_ref, o_ref):
    vs = [x_ref[0, :, k, :] for k in range(8)]
    r = vs[0].shape[0]
    while r > 1:
        h = r // 2
        vs = _merge8([v[:h] for v in vs], [v[h:] for v in vs])
        r = h
    o_ref[0] = jnp.concatenate(vs, axis=0)


def kernel(inputs):
    b, l, c = inputs.shape
    nchunk = l // _CHUNK
    part = pl.pallas_call(
        _leaf_kernel,
        grid=(b, nchunk),
        in_specs=[pl.BlockSpec((1, _CHUNK, c), lambda i, j: (i, j, 0))],
        out_specs=pl.BlockSpec((1, 1, 8, c), lambda i, j: (i, j, 0, 0)),
        out_shape=jax.ShapeDtypeStruct((b, nchunk, 8, c), inputs.dtype),
        compiler_params=pltpu.CompilerParams(
            dimension_semantics=("parallel", "parallel")),
    )(inputs)
    return pl.pallas_call(
        _combine_kernel,
        grid=(b,),
        in_specs=[pl.BlockSpec((1, nchunk, 8, c), lambda i: (i, 0, 0, 0))],
        out_specs=pl.BlockSpec((1, 8, c), lambda i: (i, 0, 0)),
        out_shape=jax.ShapeDtypeStruct((b, 8, c), inputs.dtype),
        compiler_params=pltpu.CompilerParams(
            dimension_semantics=("parallel",)),
    )(part)
